# baseline (speedup 1.0000x reference)
import functools
import math

import jax
import jax.numpy as jnp
import numpy as np
from jax.experimental import pallas as pl
from jax.experimental.pallas import tpu as pltpu

_D = 512
_H = 8
_DH = 64
_PRED = 512
_EPS = 1e-5
_INV_SQRT2 = 0.7071067811865476


def _gelu(x):
    return 0.5 * x * (1.0 + jax.lax.erf(x * _INV_SQRT2))


def _ln_rows(z, g, b):
    mu = jnp.mean(z, axis=-1, keepdims=True)
    var = jnp.mean((z - mu) ** 2, axis=-1, keepdims=True)
    return (z - mu) * jax.lax.rsqrt(var + _EPS) * g + b


def _mm_kernel(x_ref, w_ref, b_ref, o_ref):
    o_ref[...] = (
        jnp.dot(x_ref[...], w_ref[...], preferred_element_type=jnp.float32)
        + b_ref[...]
    )


def _mm(x, w, b, bm=512):
    M, K = x.shape
    N = w.shape[1]
    return pl.pallas_call(
        _mm_kernel,
        grid=(M // bm,),
        in_specs=[
            pl.BlockSpec((bm, K), lambda i: (i, 0)),
            pl.BlockSpec((K, N), lambda i: (0, 0)),
            pl.BlockSpec((1, N), lambda i: (0, 0)),
        ],
        out_specs=pl.BlockSpec((bm, N), lambda i: (i, 0)),
        out_shape=jax.ShapeDtypeStruct((M, N), jnp.float32),
        compiler_params=pltpu.CompilerParams(
            dimension_semantics=("arbitrary",)
        ),
    )(x, w, b.reshape(1, N))


def _embed_kernel(x_ref, w_ref, pe_ref, o_ref):
    o_ref[...] = (
        jnp.dot(x_ref[...], w_ref[...], preferred_element_type=jnp.float32)
        + pe_ref[...]
    )


def _embed(xcat, w, pe, bm=512):
    M, K = xcat.shape
    L = pe.shape[0]
    nlb = L // bm
    return pl.pallas_call(
        _embed_kernel,
        grid=(M // bm,),
        in_specs=[
            pl.BlockSpec((bm, K), lambda i: (i, 0)),
            pl.BlockSpec((K, _D), lambda i: (0, 0)),
            pl.BlockSpec((bm, _D), lambda i: (i % nlb, 0)),
        ],
        out_specs=pl.BlockSpec((bm, _D), lambda i: (i, 0)),
        out_shape=jax.ShapeDtypeStruct((M, _D), jnp.float32),
        compiler_params=pltpu.CompilerParams(
            dimension_semantics=("arbitrary",)
        ),
    )(xcat, w, pe)


def _attn_kernel(q_ref, k_ref, v_ref, o_ref, *, scale, causal, bq):
    q = q_ref[0]
    k = k_ref[0]
    s = jax.lax.dot_general(
        q, k, (((1,), (1,)), ((), ())), preferred_element_type=jnp.float32
    ) * scale
    if causal:
        i = pl.program_id(1)
        row = i * bq + jax.lax.broadcasted_iota(jnp.int32, s.shape, 0)
        col = jax.lax.broadcasted_iota(jnp.int32, s.shape, 1)
        s = jnp.where(col > row, jnp.float32(-1e9), s)
    m = jnp.max(s, axis=-1, keepdims=True)
    p = jnp.exp(s - m)
    l = jnp.sum(p, axis=-1, keepdims=True)
    o_ref[0] = (
        jnp.dot(p, v_ref[0], preferred_element_type=jnp.float32) / l
    )


def _attn(q, k, v, causal, bq=512):
    BH, L, DH = q.shape
    S = k.shape[1]
    scale = 1.0 / math.sqrt(DH)
    return pl.pallas_call(
        functools.partial(_attn_kernel, scale=scale, causal=causal, bq=bq),
        grid=(BH, L // bq),
        in_specs=[
            pl.BlockSpec((1, bq, DH), lambda h, i: (h, i, 0)),
            pl.BlockSpec((1, S, DH), lambda h, i: (h, 0, 0)),
            pl.BlockSpec((1, S, DH), lambda h, i: (h, 0, 0)),
        ],
        out_specs=pl.BlockSpec((1, bq, DH), lambda h, i: (h, i, 0)),
        out_shape=jax.ShapeDtypeStruct((BH, L, DH), jnp.float32),
        compiler_params=pltpu.CompilerParams(
            dimension_semantics=("arbitrary", "arbitrary")
        ),
    )(q, k, v)


def _mmo_res_ln_kernel(a_ref, w_ref, b_ref, x_ref, g_ref, bb_ref, o_ref):
    z = (
        jnp.dot(a_ref[...], w_ref[...], preferred_element_type=jnp.float32)
        + b_ref[...]
        + x_ref[...]
    )
    o_ref[...] = _ln_rows(z, g_ref[...], bb_ref[...])


def _mmo_res_ln(a, w, b, x, g, bb, bm=512):
    M, K = a.shape
    N = w.shape[1]
    return pl.pallas_call(
        _mmo_res_ln_kernel,
        grid=(M // bm,),
        in_specs=[
            pl.BlockSpec((bm, K), lambda i: (i, 0)),
            pl.BlockSpec((K, N), lambda i: (0, 0)),
            pl.BlockSpec((1, N), lambda i: (0, 0)),
            pl.BlockSpec((bm, N), lambda i: (i, 0)),
            pl.BlockSpec((1, N), lambda i: (0, 0)),
            pl.BlockSpec((1, N), lambda i: (0, 0)),
        ],
        out_specs=pl.BlockSpec((bm, N), lambda i: (i, 0)),
        out_shape=jax.ShapeDtypeStruct((M, N), jnp.float32),
        compiler_params=pltpu.CompilerParams(
            dimension_semantics=("arbitrary",)
        ),
    )(a, w, b.reshape(1, N), x, g.reshape(1, N), bb.reshape(1, N))


def _ffn_res_ln_kernel(x_ref, w1_ref, b1_ref, w2_ref, b2_ref, g_ref, bb_ref,
                       o_ref):
    x = x_ref[...]
    h = _gelu(
        jnp.dot(x, w1_ref[...], preferred_element_type=jnp.float32)
        + b1_ref[...]
    )
    z = (
        jnp.dot(h, w2_ref[...], preferred_element_type=jnp.float32)
        + b2_ref[...]
        + x
    )
    o_ref[...] = _ln_rows(z, g_ref[...], bb_ref[...])


def _ffn_res_ln(x, w1, b1, w2, b2, g, bb, bm=512):
    M, D = x.shape
    F = w1.shape[1]
    return pl.pallas_call(
        _ffn_res_ln_kernel,
        grid=(M // bm,),
        in_specs=[
            pl.BlockSpec((bm, D), lambda i: (i, 0)),
            pl.BlockSpec((D, F), lambda i: (0, 0)),
            pl.BlockSpec((1, F), lambda i: (0, 0)),
            pl.BlockSpec((F, D), lambda i: (0, 0)),
            pl.BlockSpec((1, D), lambda i: (0, 0)),
            pl.BlockSpec((1, D), lambda i: (0, 0)),
            pl.BlockSpec((1, D), lambda i: (0, 0)),
        ],
        out_specs=pl.BlockSpec((bm, D), lambda i: (i, 0)),
        out_shape=jax.ShapeDtypeStruct((M, D), jnp.float32),
        compiler_params=pltpu.CompilerParams(
            dimension_semantics=("arbitrary",)
        ),
    )(x, w1, b1.reshape(1, F), w2, b2.reshape(1, D), g.reshape(1, D),
      bb.reshape(1, D))


def _ln_kernel(x_ref, g_ref, b_ref, o_ref):
    o_ref[...] = _ln_rows(x_ref[...], g_ref[...], b_ref[...])


def _ln(x, g, b, bm=512):
    M, N = x.shape
    return pl.pallas_call(
        _ln_kernel,
        grid=(M // bm,),
        in_specs=[
            pl.BlockSpec((bm, N), lambda i: (i, 0)),
            pl.BlockSpec((1, N), lambda i: (0, 0)),
            pl.BlockSpec((1, N), lambda i: (0, 0)),
        ],
        out_specs=pl.BlockSpec((bm, N), lambda i: (i, 0)),
        out_shape=jax.ShapeDtypeStruct((M, N), jnp.float32),
        compiler_params=pltpu.CompilerParams(
            dimension_semantics=("arbitrary",)
        ),
    )(x, g.reshape(1, N), b.reshape(1, N))


def _ln_proj_kernel(x_ref, g_ref, b_ref, w_ref, wb_ref, o_ref):
    z = _ln_rows(x_ref[...], g_ref[...], b_ref[...])
    o_ref[...] = (
        jnp.dot(z, w_ref[...], preferred_element_type=jnp.float32)
        + wb_ref[...]
    )


def _ln_proj(x, g, b, wp, bp, bm=512):
    M, N = x.shape
    Np = wp.shape[1]
    return pl.pallas_call(
        _ln_proj_kernel,
        grid=(M // bm,),
        in_specs=[
            pl.BlockSpec((bm, N), lambda i: (i, 0)),
            pl.BlockSpec((1, N), lambda i: (0, 0)),
            pl.BlockSpec((1, N), lambda i: (0, 0)),
            pl.BlockSpec((N, Np), lambda i: (0, 0)),
            pl.BlockSpec((1, Np), lambda i: (0, 0)),
        ],
        out_specs=pl.BlockSpec((bm, Np), lambda i: (i, 0)),
        out_shape=jax.ShapeDtypeStruct((M, Np), jnp.float32),
        compiler_params=pltpu.CompilerParams(
            dimension_semantics=("arbitrary",)
        ),
    )(x, g.reshape(1, N), b.reshape(1, N), wp, bp.reshape(1, Np))


def _token_cat(x):
    return jnp.concatenate(
        [jnp.roll(x, 1, axis=1), x, jnp.roll(x, -1, axis=1)], axis=-1
    )


def _pos_embed_np(L, d):
    pos = np.arange(L, dtype=np.float32)[:, None]
    div = np.exp(
        np.arange(0, d, 2, dtype=np.float32) * -(np.log(10000.0) / d)
    )
    pe = np.zeros((L, d), dtype=np.float32)
    pe[:, 0::2] = np.sin(pos * div)
    pe[:, 1::2] = np.cos(pos * div)
    return jnp.asarray(pe)


def _split_heads(x, B, L):
    return (
        x.reshape(B, L, _H, _DH)
        .transpose(0, 2, 1, 3)
        .reshape(B * _H, L, _DH)
    )


def _merge_heads(o, B, L, mix):
    o = o.reshape(B, _H, L, _DH)
    if mix:
        return o.reshape(B * L, _H * _DH)
    return o.transpose(0, 2, 1, 3).reshape(B * L, _H * _DH)


def _attention_block(x2d, kv2d, p, B, L, S, causal, mix):
    wqkv = jnp.concatenate([p['Wq'], p['Wk'], p['Wv']], axis=1)
    bqkv = jnp.concatenate([p['bq'], p['bk'], p['bv']], axis=0)
    if x2d is kv2d:
        qkv = _mm(x2d, wqkv, bqkv)
        q2, k2, v2 = qkv[:, :_D], qkv[:, _D:2 * _D], qkv[:, 2 * _D:]
    else:
        q2 = _mm(x2d, p['Wq'], p['bq'])
        wkv = jnp.concatenate([p['Wk'], p['Wv']], axis=1)
        bkv = jnp.concatenate([p['bk'], p['bv']], axis=0)
        kv = _mm(kv2d, wkv, bkv)
        k2, v2 = kv[:, :_D], kv[:, _D:]
    q = _split_heads(q2, B, L)
    k = _split_heads(k2, B, S)
    v = _split_heads(v2, B, S)
    o = _attn(q, k, v, causal)
    return _merge_heads(o, B, L, mix)


def kernel(x_enc, x_dec, params):
    B, Le, Ce = x_enc.shape
    Ld = x_dec.shape[1]

    enc_cat = _token_cat(x_enc).reshape(B * Le, 3 * Ce)
    enc_cat = jnp.pad(enc_cat, ((0, 0), (0, 24 - 3 * Ce)))
    enc_w = jnp.pad(
        params['enc_tok_W'].reshape(3 * Ce, _D), ((0, 24 - 3 * Ce), (0, 0))
    )
    enc = _embed(enc_cat, enc_w, _pos_embed_np(Le, _D))

    dec_cat = _token_cat(x_dec).reshape(B * Ld, 3 * Ce)
    dec_cat = jnp.pad(dec_cat, ((0, 0), (0, 24 - 3 * Ce)))
    dec_w = jnp.pad(
        params['dec_tok_W'].reshape(3 * Ce, _D), ((0, 24 - 3 * Ce), (0, 0))
    )
    dec = _embed(dec_cat, dec_w, _pos_embed_np(Ld, _D))

    for p in params['enc_layers']:
        a = _attention_block(enc, enc, p['attn'], B, Le, Le, False, False)
        enc = _mmo_res_ln(a, p['attn']['Wo'], p['attn']['bo'], enc,
                          p['n1g'], p['n1b'])
        enc = _ffn_res_ln(enc, p['W1'], p['b1'], p['W2'], p['b2'],
                          p['n2g'], p['n2b'])
    enc = _ln(enc, params['enc_ng'], params['enc_nb'])

    for p in params['dec_layers']:
        a = _attention_block(dec, dec, p['self'], B, Ld, Ld, True, True)
        dec = _mmo_res_ln(a, p['self']['Wo'], p['self']['bo'], dec,
                          p['n1g'], p['n1b'])
        a = _attention_block(dec, enc, p['cross'], B, Ld, Le, False, False)
        dec = _mmo_res_ln(a, p['cross']['Wo'], p['cross']['bo'], dec,
                          p['n2g'], p['n2b'])
        dec = _ffn_res_ln(dec, p['W1'], p['b1'], p['W2'], p['b2'],
                          p['n3g'], p['n3b'])

    C = params['proj_W'].shape[1]
    wp = jnp.pad(params['proj_W'], ((0, 0), (0, 128 - C)))
    bp = jnp.pad(params['proj_b'], ((0, 128 - C),))
    tail = dec.reshape(B, Ld, _D)[:, -_PRED:, :].reshape(B * _PRED, _D)
    out = _ln_proj(tail, params['dec_ng'], params['dec_nb'], wp, bp)
    return out[:, :C].reshape(B, _PRED, C)


# speedup vs baseline: 2.3105x; 2.3105x over previous
import functools
import math

import jax
import jax.numpy as jnp
import numpy as np
from jax.experimental import pallas as pl
from jax.experimental.pallas import tpu as pltpu

_D = 512
_H = 8
_DH = 64
_PRED = 512
_EPS = 1e-5
_INV_SQRT2 = 0.7071067811865476


def _gelu(x):
    return 0.5 * x * (1.0 + jax.lax.erf(x * _INV_SQRT2))


def _ln_rows(z, g, b):
    mu = jnp.mean(z, axis=-1, keepdims=True)
    var = jnp.mean((z - mu) ** 2, axis=-1, keepdims=True)
    return (z - mu) * jax.lax.rsqrt(var + _EPS) * g + b


def _mm_kernel(x_ref, w_ref, b_ref, o_ref):
    o_ref[...] = (
        jnp.dot(x_ref[...], w_ref[...], preferred_element_type=jnp.float32)
        + b_ref[...]
    )


def _mm(x, w, b, bm=512):
    M, K = x.shape
    N = w.shape[1]
    return pl.pallas_call(
        _mm_kernel,
        grid=(M // bm,),
        in_specs=[
            pl.BlockSpec((bm, K), lambda i: (i, 0)),
            pl.BlockSpec((K, N), lambda i: (0, 0)),
            pl.BlockSpec((1, N), lambda i: (0, 0)),
        ],
        out_specs=pl.BlockSpec((bm, N), lambda i: (i, 0)),
        out_shape=jax.ShapeDtypeStruct((M, N), jnp.float32),
        compiler_params=pltpu.CompilerParams(
            dimension_semantics=("arbitrary",)
        ),
    )(x, w, b.reshape(1, N))


def _embed_kernel(x_ref, w_ref, pe_ref, o_ref):
    o_ref[...] = (
        jnp.dot(x_ref[...], w_ref[...], preferred_element_type=jnp.float32)
        + pe_ref[...]
    )


def _embed(xcat, w, pe, bm=512):
    M, K = xcat.shape
    L = pe.shape[0]
    nlb = L // bm
    return pl.pallas_call(
        _embed_kernel,
        grid=(M // bm,),
        in_specs=[
            pl.BlockSpec((bm, K), lambda i: (i, 0)),
            pl.BlockSpec((K, _D), lambda i: (0, 0)),
            pl.BlockSpec((bm, _D), lambda i: (i % nlb, 0)),
        ],
        out_specs=pl.BlockSpec((bm, _D), lambda i: (i, 0)),
        out_shape=jax.ShapeDtypeStruct((M, _D), jnp.float32),
        compiler_params=pltpu.CompilerParams(
            dimension_semantics=("arbitrary",)
        ),
    )(xcat, w, pe)


def _attn_heads(q, k, v, causal, scale, bq):
    outs = []
    for h in range(_H):
        qh = q[:, h * _DH:(h + 1) * _DH]
        kh = k[:, h * _DH:(h + 1) * _DH]
        vh = v[:, h * _DH:(h + 1) * _DH]
        s = jax.lax.dot_general(
            qh, kh, (((1,), (1,)), ((), ())),
            preferred_element_type=jnp.float32,
        ) * scale
        if causal:
            i = pl.program_id(1)
            row = i * bq + jax.lax.broadcasted_iota(jnp.int32, s.shape, 0)
            col = jax.lax.broadcasted_iota(jnp.int32, s.shape, 1)
            s = jnp.where(col > row, jnp.float32(-1e9), s)
        m = jnp.max(s, axis=-1, keepdims=True)
        p = jnp.exp(s - m)
        l = jnp.sum(p, axis=-1, keepdims=True)
        outs.append(
            jnp.dot(p, vh, preferred_element_type=jnp.float32) / l
        )
    return jnp.concatenate(outs, axis=-1)


def _attn_oproj_kernel(q_ref, k_ref, v_ref, wo_ref, bo_ref, x_ref, g_ref,
                       bb_ref, o_ref, *, scale, causal, bq):
    a = _attn_heads(q_ref[0], k_ref[0], v_ref[0], causal, scale, bq)
    z = (
        jnp.dot(a, wo_ref[...], preferred_element_type=jnp.float32)
        + bo_ref[...]
        + x_ref[0]
    )
    o_ref[0] = _ln_rows(z, g_ref[...], bb_ref[...])


def _attn_oproj(qarr, qcol, kvarr, kcol, vcol, wo, bo, xres, g, bb,
                causal, bq=512):
    B, L = qarr.shape[0], qarr.shape[1]
    S = kvarr.shape[1]
    scale = 1.0 / math.sqrt(_DH)
    return pl.pallas_call(
        functools.partial(_attn_oproj_kernel, scale=scale, causal=causal,
                          bq=bq),
        grid=(B, L // bq),
        in_specs=[
            pl.BlockSpec((1, bq, _D), lambda b, i: (b, i, qcol)),
            pl.BlockSpec((1, S, _D), lambda b, i: (b, 0, kcol)),
            pl.BlockSpec((1, S, _D), lambda b, i: (b, 0, vcol)),
            pl.BlockSpec((_D, _D), lambda b, i: (0, 0)),
            pl.BlockSpec((1, _D), lambda b, i: (0, 0)),
            pl.BlockSpec((1, bq, _D), lambda b, i: (b, i, 0)),
            pl.BlockSpec((1, _D), lambda b, i: (0, 0)),
            pl.BlockSpec((1, _D), lambda b, i: (0, 0)),
        ],
        out_specs=pl.BlockSpec((1, bq, _D), lambda b, i: (b, i, 0)),
        out_shape=jax.ShapeDtypeStruct((B, L, _D), jnp.float32),
        compiler_params=pltpu.CompilerParams(
            dimension_semantics=("arbitrary", "arbitrary")
        ),
    )(qarr, kvarr, kvarr, wo, bo.reshape(1, _D), xres, g.reshape(1, _D),
      bb.reshape(1, _D))


def _attn_plain_kernel(q_ref, k_ref, v_ref, o_ref, *, scale, causal, bq):
    o_ref[0] = _attn_heads(q_ref[0], k_ref[0], v_ref[0], causal, scale, bq)


def _attn_plain(qkv, causal, bq=512):
    B, L = qkv.shape[0], qkv.shape[1]
    scale = 1.0 / math.sqrt(_DH)
    return pl.pallas_call(
        functools.partial(_attn_plain_kernel, scale=scale, causal=causal,
                          bq=bq),
        grid=(B, L // bq),
        in_specs=[
            pl.BlockSpec((1, bq, _D), lambda b, i: (b, i, 0)),
            pl.BlockSpec((1, L, _D), lambda b, i: (b, 0, 1)),
            pl.BlockSpec((1, L, _D), lambda b, i: (b, 0, 2)),
        ],
        out_specs=pl.BlockSpec((1, bq, _D), lambda b, i: (b, i, 0)),
        out_shape=jax.ShapeDtypeStruct((B, L, _D), jnp.float32),
        compiler_params=pltpu.CompilerParams(
            dimension_semantics=("arbitrary", "arbitrary")
        ),
    )(qkv, qkv, qkv)


def _mmo_res_ln_kernel(a_ref, w_ref, b_ref, x_ref, g_ref, bb_ref, o_ref):
    z = (
        jnp.dot(a_ref[...], w_ref[...], preferred_element_type=jnp.float32)
        + b_ref[...]
        + x_ref[...]
    )
    o_ref[...] = _ln_rows(z, g_ref[...], bb_ref[...])


def _mmo_res_ln(a, w, b, x, g, bb, bm=512):
    M, K = a.shape
    N = w.shape[1]
    return pl.pallas_call(
        _mmo_res_ln_kernel,
        grid=(M // bm,),
        in_specs=[
            pl.BlockSpec((bm, K), lambda i: (i, 0)),
            pl.BlockSpec((K, N), lambda i: (0, 0)),
            pl.BlockSpec((1, N), lambda i: (0, 0)),
            pl.BlockSpec((bm, N), lambda i: (i, 0)),
            pl.BlockSpec((1, N), lambda i: (0, 0)),
            pl.BlockSpec((1, N), lambda i: (0, 0)),
        ],
        out_specs=pl.BlockSpec((bm, N), lambda i: (i, 0)),
        out_shape=jax.ShapeDtypeStruct((M, N), jnp.float32),
        compiler_params=pltpu.CompilerParams(
            dimension_semantics=("arbitrary",)
        ),
    )(a, w, b.reshape(1, N), x, g.reshape(1, N), bb.reshape(1, N))


def _ffn_res_ln_kernel(x_ref, w1_ref, b1_ref, w2_ref, b2_ref, g_ref, bb_ref,
                       o_ref):
    x = x_ref[...]
    h = _gelu(
        jnp.dot(x, w1_ref[...], preferred_element_type=jnp.float32)
        + b1_ref[...]
    )
    z = (
        jnp.dot(h, w2_ref[...], preferred_element_type=jnp.float32)
        + b2_ref[...]
        + x
    )
    o_ref[...] = _ln_rows(z, g_ref[...], bb_ref[...])


def _ffn_res_ln(x, w1, b1, w2, b2, g, bb, bm=512):
    M, D = x.shape
    F = w1.shape[1]
    return pl.pallas_call(
        _ffn_res_ln_kernel,
        grid=(M // bm,),
        in_specs=[
            pl.BlockSpec((bm, D), lambda i: (i, 0)),
            pl.BlockSpec((D, F), lambda i: (0, 0)),
            pl.BlockSpec((1, F), lambda i: (0, 0)),
            pl.BlockSpec((F, D), lambda i: (0, 0)),
            pl.BlockSpec((1, D), lambda i: (0, 0)),
            pl.BlockSpec((1, D), lambda i: (0, 0)),
            pl.BlockSpec((1, D), lambda i: (0, 0)),
        ],
        out_specs=pl.BlockSpec((bm, D), lambda i: (i, 0)),
        out_shape=jax.ShapeDtypeStruct((M, D), jnp.float32),
        compiler_params=pltpu.CompilerParams(
            dimension_semantics=("arbitrary",)
        ),
    )(x, w1, b1.reshape(1, F), w2, b2.reshape(1, D), g.reshape(1, D),
      bb.reshape(1, D))


def _ln_kernel(x_ref, g_ref, b_ref, o_ref):
    o_ref[...] = _ln_rows(x_ref[...], g_ref[...], b_ref[...])


def _ln(x, g, b, bm=512):
    M, N = x.shape
    return pl.pallas_call(
        _ln_kernel,
        grid=(M // bm,),
        in_specs=[
            pl.BlockSpec((bm, N), lambda i: (i, 0)),
            pl.BlockSpec((1, N), lambda i: (0, 0)),
            pl.BlockSpec((1, N), lambda i: (0, 0)),
        ],
        out_specs=pl.BlockSpec((bm, N), lambda i: (i, 0)),
        out_shape=jax.ShapeDtypeStruct((M, N), jnp.float32),
        compiler_params=pltpu.CompilerParams(
            dimension_semantics=("arbitrary",)
        ),
    )(x, g.reshape(1, N), b.reshape(1, N))


def _ln_proj_kernel(x_ref, g_ref, b_ref, w_ref, wb_ref, o_ref):
    z = _ln_rows(x_ref[...], g_ref[...], b_ref[...])
    o_ref[...] = (
        jnp.dot(z, w_ref[...], preferred_element_type=jnp.float32)
        + wb_ref[...]
    )


def _ln_proj(x, g, b, wp, bp, bm=512):
    M, N = x.shape
    Np = wp.shape[1]
    return pl.pallas_call(
        _ln_proj_kernel,
        grid=(M // bm,),
        in_specs=[
            pl.BlockSpec((bm, N), lambda i: (i, 0)),
            pl.BlockSpec((1, N), lambda i: (0, 0)),
            pl.BlockSpec((1, N), lambda i: (0, 0)),
            pl.BlockSpec((N, Np), lambda i: (0, 0)),
            pl.BlockSpec((1, Np), lambda i: (0, 0)),
        ],
        out_specs=pl.BlockSpec((bm, Np), lambda i: (i, 0)),
        out_shape=jax.ShapeDtypeStruct((M, Np), jnp.float32),
        compiler_params=pltpu.CompilerParams(
            dimension_semantics=("arbitrary",)
        ),
    )(x, g.reshape(1, N), b.reshape(1, N), wp, bp.reshape(1, Np))


def _token_cat(x):
    return jnp.concatenate(
        [jnp.roll(x, 1, axis=1), x, jnp.roll(x, -1, axis=1)], axis=-1
    )


def _pos_embed_np(L, d):
    pos = np.arange(L, dtype=np.float32)[:, None]
    div = np.exp(
        np.arange(0, d, 2, dtype=np.float32) * -(np.log(10000.0) / d)
    )
    pe = np.zeros((L, d), dtype=np.float32)
    pe[:, 0::2] = np.sin(pos * div)
    pe[:, 1::2] = np.cos(pos * div)
    return jnp.asarray(pe)


def kernel(x_enc, x_dec, params):
    B, Le, Ce = x_enc.shape
    Ld = x_dec.shape[1]

    enc_cat = _token_cat(x_enc).reshape(B * Le, 3 * Ce)
    enc_cat = jnp.pad(enc_cat, ((0, 0), (0, 24 - 3 * Ce)))
    enc_w = jnp.pad(
        params['enc_tok_W'].reshape(3 * Ce, _D), ((0, 24 - 3 * Ce), (0, 0))
    )
    enc = _embed(enc_cat, enc_w, _pos_embed_np(Le, _D))

    dec_cat = _token_cat(x_dec).reshape(B * Ld, 3 * Ce)
    dec_cat = jnp.pad(dec_cat, ((0, 0), (0, 24 - 3 * Ce)))
    dec_w = jnp.pad(
        params['dec_tok_W'].reshape(3 * Ce, _D), ((0, 24 - 3 * Ce), (0, 0))
    )
    dec = _embed(dec_cat, dec_w, _pos_embed_np(Ld, _D))

    for p in params['enc_layers']:
        pa = p['attn']
        wqkv = jnp.concatenate([pa['Wq'], pa['Wk'], pa['Wv']], axis=1)
        bqkv = jnp.concatenate([pa['bq'], pa['bk'], pa['bv']], axis=0)
        qkv = _mm(enc, wqkv, bqkv).reshape(B, Le, 3 * _D)
        enc = _attn_oproj(qkv, 0, qkv, 1, 2, pa['Wo'], pa['bo'],
                          enc.reshape(B, Le, _D), p['n1g'], p['n1b'],
                          causal=False).reshape(B * Le, _D)
        enc = _ffn_res_ln(enc, p['W1'], p['b1'], p['W2'], p['b2'],
                          p['n2g'], p['n2b'])
    enc = _ln(enc, params['enc_ng'], params['enc_nb'])

    for p in params['dec_layers']:
        ps = p['self']
        wqkv = jnp.concatenate([ps['Wq'], ps['Wk'], ps['Wv']], axis=1)
        bqkv = jnp.concatenate([ps['bq'], ps['bk'], ps['bv']], axis=0)
        qkv = _mm(dec, wqkv, bqkv).reshape(B, Ld, 3 * _D)
        a = _attn_plain(qkv, causal=True)
        a = (a.reshape(B, Ld, _H, _DH).transpose(0, 2, 1, 3)
             .reshape(B * Ld, _D))
        dec = _mmo_res_ln(a, ps['Wo'], ps['bo'], dec, p['n1g'], p['n1b'])

        pc = p['cross']
        q2 = _mm(dec, pc['Wq'], pc['bq']).reshape(B, Ld, _D)
        wkv = jnp.concatenate([pc['Wk'], pc['Wv']], axis=1)
        bkv = jnp.concatenate([pc['bk'], pc['bv']], axis=0)
        kv = _mm(enc, wkv, bkv).reshape(B, Le, 2 * _D)
        dec = _attn_oproj(q2, 0, kv, 0, 1, pc['Wo'], pc['bo'],
                          dec.reshape(B, Ld, _D), p['n2g'], p['n2b'],
                          causal=False).reshape(B * Ld, _D)
        dec = _ffn_res_ln(dec, p['W1'], p['b1'], p['W2'], p['b2'],
                          p['n3g'], p['n3b'])

    C = params['proj_W'].shape[1]
    wp = jnp.pad(params['proj_W'], ((0, 0), (0, 128 - C)))
    bp = jnp.pad(params['proj_b'], ((0, 128 - C),))
    tail = dec.reshape(B, Ld, _D)[:, -_PRED:, :].reshape(B * _PRED, _D)
    out = _ln_proj(tail, params['dec_ng'], params['dec_nb'], wp, bp)
    return out[:, :C].reshape(B, _PRED, C)


# speedup vs baseline: 2.4033x; 1.0402x over previous
import functools
import math

import jax
import jax.numpy as jnp
import numpy as np
from jax.experimental import pallas as pl
from jax.experimental.pallas import tpu as pltpu

_D = 512
_H = 8
_DH = 64
_PRED = 512
_EPS = 1e-5
_INV_SQRT2 = 0.7071067811865476


def _gelu(x):
    return 0.5 * x * (1.0 + jax.lax.erf(x * _INV_SQRT2))


def _ln_rows(z, g, b):
    mu = jnp.mean(z, axis=-1, keepdims=True)
    var = jnp.mean((z - mu) ** 2, axis=-1, keepdims=True)
    return (z - mu) * jax.lax.rsqrt(var + _EPS) * g + b


_BF = jnp.bfloat16


def _mm_kernel(x_ref, w_ref, b_ref, o_ref):
    y = (
        jnp.dot(x_ref[...].astype(_BF), w_ref[...],
                preferred_element_type=jnp.float32)
        + b_ref[...]
    )
    o_ref[...] = y.astype(o_ref.dtype)


def _mm(x, w, b, bm=512, out_dtype=jnp.float32):
    M, K = x.shape
    N = w.shape[1]
    return pl.pallas_call(
        _mm_kernel,
        grid=(M // bm,),
        in_specs=[
            pl.BlockSpec((bm, K), lambda i: (i, 0)),
            pl.BlockSpec((K, N), lambda i: (0, 0)),
            pl.BlockSpec((1, N), lambda i: (0, 0)),
        ],
        out_specs=pl.BlockSpec((bm, N), lambda i: (i, 0)),
        out_shape=jax.ShapeDtypeStruct((M, N), out_dtype),
        compiler_params=pltpu.CompilerParams(
            dimension_semantics=("arbitrary",)
        ),
    )(x, w, b.reshape(1, N))


def _embed_kernel(x_ref, w_ref, pe_ref, o_ref):
    o_ref[...] = (
        jnp.dot(x_ref[...], w_ref[...], preferred_element_type=jnp.float32)
        + pe_ref[...]
    )


def _embed(xcat, w, pe, bm=512):
    M, K = xcat.shape
    L = pe.shape[0]
    nlb = L // bm
    return pl.pallas_call(
        _embed_kernel,
        grid=(M // bm,),
        in_specs=[
            pl.BlockSpec((bm, K), lambda i: (i, 0)),
            pl.BlockSpec((K, _D), lambda i: (0, 0)),
            pl.BlockSpec((bm, _D), lambda i: (i % nlb, 0)),
        ],
        out_specs=pl.BlockSpec((bm, _D), lambda i: (i, 0)),
        out_shape=jax.ShapeDtypeStruct((M, _D), jnp.float32),
        compiler_params=pltpu.CompilerParams(
            dimension_semantics=("arbitrary",)
        ),
    )(xcat, w, pe)


def _attn_heads(q, k, v, causal, scale, bq):
    outs = []
    for h in range(_H):
        qh = q[:, h * _DH:(h + 1) * _DH]
        kh = k[:, h * _DH:(h + 1) * _DH]
        vh = v[:, h * _DH:(h + 1) * _DH]
        s = jax.lax.dot_general(
            qh, kh, (((1,), (1,)), ((), ())),
            preferred_element_type=jnp.float32,
        ) * scale
        if causal:
            i = pl.program_id(1)
            row = i * bq + jax.lax.broadcasted_iota(jnp.int32, s.shape, 0)
            col = jax.lax.broadcasted_iota(jnp.int32, s.shape, 1)
            s = jnp.where(col > row, jnp.float32(-1e9), s)
        m = jnp.max(s, axis=-1, keepdims=True)
        p = jnp.exp(s - m)
        l = jnp.sum(p, axis=-1, keepdims=True)
        o = jnp.dot(p.astype(_BF), vh, preferred_element_type=jnp.float32)
        outs.append(o / l)
    return jnp.concatenate(outs, axis=-1)


def _attn_oproj_kernel(q_ref, k_ref, v_ref, wo_ref, bo_ref, x_ref, g_ref,
                       bb_ref, o_ref, *, scale, causal, bq):
    a = _attn_heads(q_ref[0], k_ref[0], v_ref[0], causal, scale, bq)
    z = (
        jnp.dot(a.astype(_BF), wo_ref[...],
                preferred_element_type=jnp.float32)
        + bo_ref[...]
        + x_ref[0]
    )
    o_ref[0] = _ln_rows(z, g_ref[...], bb_ref[...])


def _attn_oproj(qarr, qcol, kvarr, kcol, vcol, wo, bo, xres, g, bb,
                causal, bq=512):
    B, L = qarr.shape[0], qarr.shape[1]
    S = kvarr.shape[1]
    scale = 1.0 / math.sqrt(_DH)
    return pl.pallas_call(
        functools.partial(_attn_oproj_kernel, scale=scale, causal=causal,
                          bq=bq),
        grid=(B, L // bq),
        in_specs=[
            pl.BlockSpec((1, bq, _D), lambda b, i: (b, i, qcol)),
            pl.BlockSpec((1, S, _D), lambda b, i: (b, 0, kcol)),
            pl.BlockSpec((1, S, _D), lambda b, i: (b, 0, vcol)),
            pl.BlockSpec((_D, _D), lambda b, i: (0, 0)),
            pl.BlockSpec((1, _D), lambda b, i: (0, 0)),
            pl.BlockSpec((1, bq, _D), lambda b, i: (b, i, 0)),
            pl.BlockSpec((1, _D), lambda b, i: (0, 0)),
            pl.BlockSpec((1, _D), lambda b, i: (0, 0)),
        ],
        out_specs=pl.BlockSpec((1, bq, _D), lambda b, i: (b, i, 0)),
        out_shape=jax.ShapeDtypeStruct((B, L, _D), jnp.float32),
        compiler_params=pltpu.CompilerParams(
            dimension_semantics=("arbitrary", "arbitrary")
        ),
    )(qarr, kvarr, kvarr, wo, bo.reshape(1, _D), xres, g.reshape(1, _D),
      bb.reshape(1, _D))


def _attn_plain_kernel(q_ref, k_ref, v_ref, o_ref, *, scale, causal, bq):
    a = _attn_heads(q_ref[0], k_ref[0], v_ref[0], causal, scale, bq)
    o_ref[0] = a.astype(o_ref.dtype)


def _attn_plain(qkv, causal, bq=512):
    B, L = qkv.shape[0], qkv.shape[1]
    scale = 1.0 / math.sqrt(_DH)
    return pl.pallas_call(
        functools.partial(_attn_plain_kernel, scale=scale, causal=causal,
                          bq=bq),
        grid=(B, L // bq),
        in_specs=[
            pl.BlockSpec((1, bq, _D), lambda b, i: (b, i, 0)),
            pl.BlockSpec((1, L, _D), lambda b, i: (b, 0, 1)),
            pl.BlockSpec((1, L, _D), lambda b, i: (b, 0, 2)),
        ],
        out_specs=pl.BlockSpec((1, bq, _D), lambda b, i: (b, i, 0)),
        out_shape=jax.ShapeDtypeStruct((B, L, _D), _BF),
        compiler_params=pltpu.CompilerParams(
            dimension_semantics=("arbitrary", "arbitrary")
        ),
    )(qkv, qkv, qkv)


def _mmo_res_ln_kernel(a_ref, w_ref, b_ref, x_ref, g_ref, bb_ref, o_ref):
    z = (
        jnp.dot(a_ref[...].astype(_BF), w_ref[...],
                preferred_element_type=jnp.float32)
        + b_ref[...]
        + x_ref[...]
    )
    o_ref[...] = _ln_rows(z, g_ref[...], bb_ref[...])


def _mmo_res_ln(a, w, b, x, g, bb, bm=512):
    M, K = a.shape
    N = w.shape[1]
    return pl.pallas_call(
        _mmo_res_ln_kernel,
        grid=(M // bm,),
        in_specs=[
            pl.BlockSpec((bm, K), lambda i: (i, 0)),
            pl.BlockSpec((K, N), lambda i: (0, 0)),
            pl.BlockSpec((1, N), lambda i: (0, 0)),
            pl.BlockSpec((bm, N), lambda i: (i, 0)),
            pl.BlockSpec((1, N), lambda i: (0, 0)),
            pl.BlockSpec((1, N), lambda i: (0, 0)),
        ],
        out_specs=pl.BlockSpec((bm, N), lambda i: (i, 0)),
        out_shape=jax.ShapeDtypeStruct((M, N), jnp.float32),
        compiler_params=pltpu.CompilerParams(
            dimension_semantics=("arbitrary",)
        ),
    )(a, w, b.reshape(1, N), x, g.reshape(1, N), bb.reshape(1, N))


def _ffn_res_ln_kernel(x_ref, w1_ref, b1_ref, w2_ref, b2_ref, g_ref, bb_ref,
                       o_ref):
    x = x_ref[...]
    h = _gelu(
        jnp.dot(x.astype(_BF), w1_ref[...],
                preferred_element_type=jnp.float32)
        + b1_ref[...]
    )
    z = (
        jnp.dot(h.astype(_BF), w2_ref[...],
                preferred_element_type=jnp.float32)
        + b2_ref[...]
        + x
    )
    o_ref[...] = _ln_rows(z, g_ref[...], bb_ref[...])


def _ffn_res_ln(x, w1, b1, w2, b2, g, bb, bm=512):
    M, D = x.shape
    F = w1.shape[1]
    return pl.pallas_call(
        _ffn_res_ln_kernel,
        grid=(M // bm,),
        in_specs=[
            pl.BlockSpec((bm, D), lambda i: (i, 0)),
            pl.BlockSpec((D, F), lambda i: (0, 0)),
            pl.BlockSpec((1, F), lambda i: (0, 0)),
            pl.BlockSpec((F, D), lambda i: (0, 0)),
            pl.BlockSpec((1, D), lambda i: (0, 0)),
            pl.BlockSpec((1, D), lambda i: (0, 0)),
            pl.BlockSpec((1, D), lambda i: (0, 0)),
        ],
        out_specs=pl.BlockSpec((bm, D), lambda i: (i, 0)),
        out_shape=jax.ShapeDtypeStruct((M, D), jnp.float32),
        compiler_params=pltpu.CompilerParams(
            dimension_semantics=("arbitrary",)
        ),
    )(x, w1, b1.reshape(1, F), w2, b2.reshape(1, D), g.reshape(1, D),
      bb.reshape(1, D))


def _ln_kernel(x_ref, g_ref, b_ref, o_ref):
    o_ref[...] = _ln_rows(x_ref[...], g_ref[...], b_ref[...])


def _ln(x, g, b, bm=512):
    M, N = x.shape
    return pl.pallas_call(
        _ln_kernel,
        grid=(M // bm,),
        in_specs=[
            pl.BlockSpec((bm, N), lambda i: (i, 0)),
            pl.BlockSpec((1, N), lambda i: (0, 0)),
            pl.BlockSpec((1, N), lambda i: (0, 0)),
        ],
        out_specs=pl.BlockSpec((bm, N), lambda i: (i, 0)),
        out_shape=jax.ShapeDtypeStruct((M, N), jnp.float32),
        compiler_params=pltpu.CompilerParams(
            dimension_semantics=("arbitrary",)
        ),
    )(x, g.reshape(1, N), b.reshape(1, N))


def _ln_proj_kernel(x_ref, g_ref, b_ref, w_ref, wb_ref, o_ref):
    z = _ln_rows(x_ref[...], g_ref[...], b_ref[...])
    o_ref[...] = (
        jnp.dot(z, w_ref[...], preferred_element_type=jnp.float32)
        + wb_ref[...]
    )


def _ln_proj(x, g, b, wp, bp, bm=512):
    M, N = x.shape
    Np = wp.shape[1]
    return pl.pallas_call(
        _ln_proj_kernel,
        grid=(M // bm,),
        in_specs=[
            pl.BlockSpec((bm, N), lambda i: (i, 0)),
            pl.BlockSpec((1, N), lambda i: (0, 0)),
            pl.BlockSpec((1, N), lambda i: (0, 0)),
            pl.BlockSpec((N, Np), lambda i: (0, 0)),
            pl.BlockSpec((1, Np), lambda i: (0, 0)),
        ],
        out_specs=pl.BlockSpec((bm, Np), lambda i: (i, 0)),
        out_shape=jax.ShapeDtypeStruct((M, Np), jnp.float32),
        compiler_params=pltpu.CompilerParams(
            dimension_semantics=("arbitrary",)
        ),
    )(x, g.reshape(1, N), b.reshape(1, N), wp, bp.reshape(1, Np))


def _token_cat(x):
    return jnp.concatenate(
        [jnp.roll(x, 1, axis=1), x, jnp.roll(x, -1, axis=1)], axis=-1
    )


def _pos_embed_np(L, d):
    pos = np.arange(L, dtype=np.float32)[:, None]
    div = np.exp(
        np.arange(0, d, 2, dtype=np.float32) * -(np.log(10000.0) / d)
    )
    pe = np.zeros((L, d), dtype=np.float32)
    pe[:, 0::2] = np.sin(pos * div)
    pe[:, 1::2] = np.cos(pos * div)
    return jnp.asarray(pe)


def kernel(x_enc, x_dec, params):
    B, Le, Ce = x_enc.shape
    Ld = x_dec.shape[1]

    enc_cat = _token_cat(x_enc).reshape(B * Le, 3 * Ce)
    enc_cat = jnp.pad(enc_cat, ((0, 0), (0, 24 - 3 * Ce)))
    enc_w = jnp.pad(
        params['enc_tok_W'].reshape(3 * Ce, _D), ((0, 24 - 3 * Ce), (0, 0))
    )
    enc = _embed(enc_cat, enc_w, _pos_embed_np(Le, _D))

    dec_cat = _token_cat(x_dec).reshape(B * Ld, 3 * Ce)
    dec_cat = jnp.pad(dec_cat, ((0, 0), (0, 24 - 3 * Ce)))
    dec_w = jnp.pad(
        params['dec_tok_W'].reshape(3 * Ce, _D), ((0, 24 - 3 * Ce), (0, 0))
    )
    dec = _embed(dec_cat, dec_w, _pos_embed_np(Ld, _D))

    for p in params['enc_layers']:
        pa = p['attn']
        wqkv = jnp.concatenate([pa['Wq'], pa['Wk'], pa['Wv']],
                               axis=1).astype(_BF)
        bqkv = jnp.concatenate([pa['bq'], pa['bk'], pa['bv']], axis=0)
        qkv = _mm(enc, wqkv, bqkv, out_dtype=_BF).reshape(B, Le, 3 * _D)
        enc = _attn_oproj(qkv, 0, qkv, 1, 2, pa['Wo'].astype(_BF),
                          pa['bo'], enc.reshape(B, Le, _D),
                          p['n1g'], p['n1b'],
                          causal=False).reshape(B * Le, _D)
        enc = _ffn_res_ln(enc, p['W1'].astype(_BF), p['b1'],
                          p['W2'].astype(_BF), p['b2'],
                          p['n2g'], p['n2b'])
    enc = _ln(enc, params['enc_ng'], params['enc_nb'])

    for p in params['dec_layers']:
        ps = p['self']
        wqkv = jnp.concatenate([ps['Wq'], ps['Wk'], ps['Wv']],
                               axis=1).astype(_BF)
        bqkv = jnp.concatenate([ps['bq'], ps['bk'], ps['bv']], axis=0)
        qkv = _mm(dec, wqkv, bqkv, out_dtype=_BF).reshape(B, Ld, 3 * _D)
        a = _attn_plain(qkv, causal=True)
        a = (a.reshape(B, Ld, _H, _DH).transpose(0, 2, 1, 3)
             .reshape(B * Ld, _D))
        dec = _mmo_res_ln(a, ps['Wo'].astype(_BF), ps['bo'], dec,
                          p['n1g'], p['n1b'])

        pc = p['cross']
        q2 = _mm(dec, pc['Wq'].astype(_BF), pc['bq'],
                 out_dtype=_BF).reshape(B, Ld, _D)
        wkv = jnp.concatenate([pc['Wk'], pc['Wv']], axis=1).astype(_BF)
        bkv = jnp.concatenate([pc['bk'], pc['bv']], axis=0)
        kv = _mm(enc, wkv, bkv, out_dtype=_BF).reshape(B, Le, 2 * _D)
        dec = _attn_oproj(q2, 0, kv, 0, 1, pc['Wo'].astype(_BF), pc['bo'],
                          dec.reshape(B, Ld, _D), p['n2g'], p['n2b'],
                          causal=False).reshape(B * Ld, _D)
        dec = _ffn_res_ln(dec, p['W1'].astype(_BF), p['b1'],
                          p['W2'].astype(_BF), p['b2'],
                          p['n3g'], p['n3b'])

    C = params['proj_W'].shape[1]
    wp = jnp.pad(params['proj_W'], ((0, 0), (0, 128 - C)))
    bp = jnp.pad(params['proj_b'], ((0, 128 - C),))
    tail = dec.reshape(B, Ld, _D)[:, -_PRED:, :].reshape(B * _PRED, _D)
    out = _ln_proj(tail, params['dec_ng'], params['dec_nb'], wp, bp)
    return out[:, :C].reshape(B, _PRED, C)
